# Initial kernel scaffold; baseline (speedup 1.0000x reference)
import jax
import jax.numpy as jnp
from jax.experimental import pallas as pl


def kernel(sequence, topic_tables, shared_table):
    raise NotImplementedError("write your pallas kernel here")


# speedup vs baseline: 4.0072x; 4.0072x over previous
import functools

import jax
import jax.numpy as jnp
from jax import lax
from jax.experimental import pallas as pl
from jax.experimental.pallas import tpu as pltpu
from jax.experimental.pallas import tpu_sc as plsc

NUM_TOPICS = 8
FT = 32
FS = 16
L = 200
LH = 100


@functools.lru_cache(maxsize=None)
def _make_kernel(B, V):
    info = plsc.get_sparse_core_info()
    NC, NS = info.num_cores, info.num_subcores
    NW = NC * NS
    assert B % NW == 0
    b_per_w = B // NW

    mesh = plsc.VectorSubcoreMesh(core_axis_name="c", subcore_axis_name="s")
    out_type = tuple(
        jax.ShapeDtypeStruct((B, FT + FS, L), jnp.float32)
        for _ in range(NUM_TOPICS)
    )

    @functools.partial(
        pl.kernel,
        mesh=mesh,
        out_type=out_type,
        compiler_params=pltpu.CompilerParams(
            use_tc_tiling_on_sc=False, needs_layout_passes=False),
        scratch_types=[
            pltpu.VMEM((2, LH), jnp.int32),
            pltpu.VMEM((NUM_TOPICS, L, FT), jnp.float32),
            pltpu.VMEM((L, FS), jnp.float32),
            pltpu.VMEM((NUM_TOPICS, FT, L), jnp.float32),
            pltpu.VMEM((FS, L), jnp.float32),
            pltpu.SemaphoreType.DMA,
        ],
    )
    def topic_kernel(seq_hbm, topics_hbm, shared_hbm, *rest):
        outs = rest[:NUM_TOPICS]
        idx_v, rows_v, sh_rows_v, tb_v, sh_t_v, sem = rest[NUM_TOPICS:]

        wid = lax.axis_index("c") * NS + lax.axis_index("s")
        b0 = wid * b_per_w
        f16 = jnp.arange(16, dtype=jnp.int32)

        def b_body(bi, carry):
            b = b0 + bi
            pltpu.sync_copy(seq_hbm.at[b], idx_v)

            copies = []
            for i in range(NUM_TOPICS):
                for h in range(2):
                    copies.append(pltpu.async_copy(
                        topics_hbm.at[i].at[idx_v.at[h]],
                        rows_v.at[i, pl.ds(h * LH, LH)],
                        sem,
                    ))
            for h in range(2):
                copies.append(pltpu.async_copy(
                    shared_hbm.at[idx_v.at[h]],
                    sh_rows_v.at[pl.ds(h * LH, LH)],
                    sem,
                ))
            for c in copies:
                c.wait()

            def l_body(l, c2):
                lv = jnp.full((16,), l, dtype=jnp.int32)
                for i in range(NUM_TOPICS):
                    iv = jnp.full((16,), i, dtype=jnp.int32)
                    x0 = rows_v[i, l, pl.ds(0, 16)]
                    plsc.store_scatter(tb_v, [iv, f16, lv], x0)
                    x1 = rows_v[i, l, pl.ds(16, 16)]
                    plsc.store_scatter(tb_v, [iv, f16 + 16, lv], x1)
                xs = sh_rows_v[l, pl.ds(0, 16)]
                plsc.store_scatter(sh_t_v, [f16, lv], xs)
                return c2

            lax.fori_loop(0, L, l_body, 0)

            for i in range(NUM_TOPICS):
                pltpu.sync_copy(tb_v.at[i], outs[i].at[b, pl.ds(0, FT)])
                pltpu.sync_copy(sh_t_v, outs[i].at[b, pl.ds(FT, FS)])
            return carry

        lax.fori_loop(0, b_per_w, b_body, 0)

    return topic_kernel


def kernel(sequence, topic_tables, shared_table):
    B = sequence.shape[0]
    V = topic_tables.shape[1]
    seq = sequence.astype(jnp.int32).reshape(B, 2, LH)
    return _make_kernel(B, V)(seq, topic_tables, shared_table)


# speedup vs baseline: 7.4253x; 1.8530x over previous
import functools

import jax
import jax.numpy as jnp
from jax import lax
from jax.experimental import pallas as pl
from jax.experimental.pallas import tpu as pltpu
from jax.experimental.pallas import tpu_sc as plsc

NUM_TOPICS = 8
FT = 32
FS = 16
L = 200
B = 1024
LT = L // 8
BT = B // 128
UNITS = LT * BT


@functools.lru_cache(maxsize=None)
def _make_kernel(V):
    info = plsc.get_sparse_core_info()
    NC, NS = info.num_cores, info.num_subcores
    NW = NC * NS
    units_per_w = -(-UNITS // NW)

    mesh = plsc.VectorSubcoreMesh(core_axis_name="c", subcore_axis_name="s")
    out_type = tuple(
        jax.ShapeDtypeStruct((FT + FS, LT, BT, 8, 128), jnp.float32)
        for _ in range(NUM_TOPICS)
    )

    @functools.partial(
        pl.kernel,
        mesh=mesh,
        out_type=out_type,
        compiler_params=pltpu.CompilerParams(
            use_tc_tiling_on_sc=False, needs_layout_passes=False),
        scratch_types=[
            pltpu.VMEM((8, 128), jnp.int32),
            pltpu.VMEM((2, 4, 128, FT), jnp.float32),
            pltpu.VMEM((2, FT, 4, 128), jnp.float32),
            pltpu.VMEM((2, 4, 128, FS), jnp.float32),
            pltpu.VMEM((2, FS, 4, 128), jnp.float32),
            pltpu.SemaphoreType.DMA,
            pltpu.SemaphoreType.DMA,
        ],
    )
    def topic_kernel(seq_hbm, topics_hbm, shared_hbm, *rest):
        outs = rest[:NUM_TOPICS]
        idx_v, rows_v, t_v, sh_rows_v, sh_t_v, sem_g, sem_o = rest[NUM_TOPICS:]

        wid = lax.axis_index("c") * NS + lax.axis_index("s")
        iota16 = jnp.arange(16, dtype=jnp.int32)
        rot = [((iota16 + j) & 15) for j in range(16)]

        steps = [(i, h) for i in range(NUM_TOPICS) for h in range(2)]
        steps += [(NUM_TOPICS, 0), (NUM_TOPICS, 1)]

        def transpose_block(rows_ref, t_ref, nf0):
            def body(q, c):
                l8 = q // (nf0 * 8)
                r = q - l8 * (nf0 * 8)
                f0 = r // 8
                b0 = (r - f0 * 8) * 16
                lv = jnp.full((16,), l8, jnp.int32)
                fv = iota16 + f0 * 16
                for j in range(16):
                    bv = rot[j] + b0
                    x = plsc.load_gather(rows_ref, [lv, bv, fv])
                    plsc.store_scatter(t_ref, [fv, lv, bv], x)
                return c
            lax.fori_loop(0, 4 * nf0 * 8, body, 0, unroll=2)

        def unit_body(r, carry):
            u = wid + r * NW
            valid = u < UNITS

            @pl.when(valid)
            def _():
                ult = u // BT
                ubt = u - ult * BT
                pltpu.sync_copy(seq_hbm.at[ult, ubt], idx_v)

                def fire(s):
                    i, h = steps[s]
                    p = s & 1
                    cs = []
                    for l in range(4):
                        la = h * 4 + l
                        if i < NUM_TOPICS:
                            cs.append(pltpu.async_copy(
                                topics_hbm.at[i].at[idx_v.at[la]],
                                rows_v.at[p, l], sem_g))
                        else:
                            cs.append(pltpu.async_copy(
                                shared_hbm.at[idx_v.at[la]],
                                sh_rows_v.at[p, l], sem_g))
                    return cs

                pending_g = fire(0)
                pending_o = [[], []]
                for s in range(len(steps)):
                    i, h = steps[s]
                    p = s & 1
                    nxt = fire(s + 1) if s + 1 < len(steps) else []
                    for c in pending_g:
                        c.wait()
                    for c in pending_o[p]:
                        c.wait()
                    if i < NUM_TOPICS:
                        transpose_block(rows_v.at[p], t_v.at[p], 2)
                        pending_o[p] = [pltpu.async_copy(
                            t_v.at[p],
                            outs[i].at[pl.ds(0, FT), ult, ubt, pl.ds(h * 4, 4)],
                            sem_o)]
                    else:
                        transpose_block(sh_rows_v.at[p], sh_t_v.at[p], 1)
                        pending_o[p] = [pltpu.async_copy(
                            sh_t_v.at[p],
                            outs[k].at[pl.ds(FT, FS), ult, ubt, pl.ds(h * 4, 4)],
                            sem_o) for k in range(NUM_TOPICS)]
                    pending_g = nxt
                for ps in pending_o:
                    for c in ps:
                        c.wait()
            return carry

        lax.fori_loop(0, units_per_w, unit_body, 0)

    return topic_kernel


def kernel(sequence, topic_tables, shared_table):
    V = topic_tables.shape[1]
    seq4 = (sequence.astype(jnp.int32)
            .reshape(BT, 128, LT, 8).transpose(2, 0, 3, 1))
    outs = _make_kernel(V)(seq4, topic_tables, shared_table)
    return tuple(
        o.transpose(2, 4, 0, 1, 3).reshape(B, FT + FS, L) for o in outs
    )
